# baseline (speedup 1.0000x reference)
import functools

import jax
import jax.numpy as jnp
from jax import lax
from jax.experimental import pallas as pl
from jax.experimental.pallas import tpu as pltpu
from jax.experimental.pallas import tpu_sc as plsc

_N = 10000
_E = 640000
_NUM_OBJ = 151
_EMBED = 200
_ACC_W = 144
_NACC = 10016
_CH = 80
_NC = 2
_NS = 16
_NW = _NC * _NS
_NCHUNK = 256
_IB = 4
_EPW = _NCHUNK * _CH
_EP = _EPW * _NW
_RPT = _NACC // _NS
_BN = 1000
_BIG = 1e30


def _lrelu(x):
    return jnp.where(x >= 0.0, x, 0.2 * x)


def _vgather(x, idx):
    dn = lax.GatherDimensionNumbers(offset_dims=(), collapsed_slice_dims=(0,),
                                    start_index_map=(0,))
    return lax.gather(x, idx[:, None], dn, slice_sizes=(1,),
                      mode=lax.GatherScatterMode.PROMISE_IN_BOUNDS)


def _dense_pre_body(roi, lbl, pos, c1w, emb1p, w1, b1, w2, b2, ms, md,
                    pmat, qmat, xw_o, sdd_o, dist_o, bnd_o):
    i = pl.program_id(0)
    nsteps = pl.num_programs(0)
    lblv = lbl[...]
    iota = lax.broadcasted_iota(jnp.int32, (_BN, 160), 1)
    p = (iota == lblv).astype(jnp.float32)
    dist_o[...] = p[:, :_NUM_OBJ] * 2000.0 - 1000.0
    h = jnp.maximum(pos[...] @ w1[...] + b1[...], 0.0)
    pe = jnp.maximum(h @ w2[...] + b2[...], 0.0)
    cw = c1w[...]
    m1 = emb1p[...] @ cw[128:328]
    xw = roi[...] @ cw[:128] + p @ m1 + pe @ cw[328:456]
    sds = xw @ ms[...]
    sdd = xw @ md[...]
    xw_o[...] = jnp.concatenate(
        [xw @ pmat[...], sds @ qmat[...]], axis=1).astype(jnp.bfloat16)
    sdd_o[...] = sdd
    v = jnp.concatenate([jnp.max(sds, axis=0, keepdims=True)[:, :8],
                         jnp.max(sdd, axis=0, keepdims=True)[:, :8]], axis=1)
    vb = jnp.broadcast_to(v, (8, 16))

    @pl.when(i == 0)
    def _():
        bnd_o[...] = jnp.full((8, 16), -_BIG, jnp.float32)

    bnd_o[...] = jnp.maximum(bnd_o[...], vb)

    @pl.when(i == nsteps - 1)
    def _():
        cur = bnd_o[...]
        ssum = cur[:, :8] + cur[:, 8:16]
        b = _lrelu(ssum)
        bnd_o[...] = jnp.concatenate(
            [b, jnp.full((8, 8), _BIG, jnp.float32)], axis=1)


def _dense_pre(roi, lbl2, posp, c1w, emb1p, w1p, b1r, w2r, b2r, ms, md,
               pmat, qmat):
    nb = _N // _BN
    full = lambda shape: pl.BlockSpec(shape, lambda i: (0, 0))
    blk = lambda w: pl.BlockSpec((_BN, w), lambda i: (i, 0))
    return pl.pallas_call(
        _dense_pre_body,
        grid=(nb,),
        in_specs=[blk(128), blk(1), blk(16), full((456, 128)),
                  full((160, 128 + 72)), full((16, 32)), full((1, 32)),
                  full((32, 128)), full((1, 128)), full((128, 16)),
                  full((128, 16)), full((128, 128)), full((16, 32))],
        out_specs=[blk(160), blk(16), blk(_NUM_OBJ),
                   pl.BlockSpec((8, 16), lambda i: (0, 0))],
        out_shape=[
            jax.ShapeDtypeStruct((_N, 160), jnp.bfloat16),
            jax.ShapeDtypeStruct((_N, 16), jnp.float32),
            jax.ShapeDtypeStruct((_N, _NUM_OBJ), jnp.float32),
            jax.ShapeDtypeStruct((8, 16), jnp.float32),
        ],
    )(roi, lbl2, posp, c1w, emb1p, w1p, b1r, w2r, b2r, ms, md, pmat, qmat)


def _edge_body(H, src_hbm, dst_hbm, xw_hbm, sdd_hbm, bnd_hbm,
               zeros_hbm, out_hbm, srcB, dstB, gd2, rows2, msg2,
               bnd_v, acc_sh, gsem0, gsem1, ssem0, ssem1, isem):
    c = lax.axis_index("c")
    s = lax.axis_index("s")
    wid = c * _NS + s
    base = wid * _EPW
    gsems = [gsem0, gsem1]
    ssems = [ssem0, ssem1]
    pltpu.sync_copy(zeros_hbm.at[pl.ds(s * _RPT, _RPT)],
                    acc_sh.at[pl.ds(s * _RPT, _RPT)])
    pltpu.sync_copy(bnd_hbm, bnd_v)
    plsc.subcore_barrier()
    bvec = bnd_v[...]

    def fire_idx_block(bb, blk):
        row0 = wid * _NCHUNK + blk * _IB
        pltpu.async_copy(src_hbm.at[pl.ds(row0, _IB)], srcB.at[bb], isem)
        pltpu.async_copy(dst_hbm.at[pl.ds(row0, _IB)], dstB.at[bb], isem)

    def drain_idx_block(bb):
        pltpu.make_async_copy(src_hbm.at[pl.ds(0, _IB)], srcB.at[bb],
                              isem).wait()
        pltpu.make_async_copy(dst_hbm.at[pl.ds(0, _IB)], dstB.at[bb],
                              isem).wait()

    def fire_gathers(b, sidx, didx):
        pltpu.async_copy(xw_hbm.at[sidx], rows2.at[b], gsems[b])
        pltpu.async_copy(sdd_hbm.at[didx], gd2.at[b], gsems[b])

    def drain_gathers(b):
        pltpu.make_async_copy(xw_hbm.at[pl.ds(0, _CH)], rows2.at[b],
                              gsems[b]).wait()
        pltpu.make_async_copy(sdd_hbm.at[pl.ds(0, _CH)], gd2.at[b],
                              gsems[b]).wait()

    def drain_scatter(b):
        pltpu.make_async_copy(zeros_hbm.at[pl.ds(0, _CH)], msg2.at[b],
                              ssems[b]).wait()

    def compute_chunk(b):
        himask = jnp.full((16,), -65536, jnp.int32)

        @plsc.parallel_loop(0, _CH, step=1, unroll=4)
        def edge_body(e):
            rrow = rows2.at[b].at[e]
            wsd = rrow[pl.ds(64, 16)]
            gs = lax.bitcast_convert_type(wsd << 16, jnp.float32)
            a = gs + gd2.at[b][e]
            ex = jnp.exp(_lrelu(a) - bvec)
            if H == 1:
                exb = _vgather(ex, jnp.zeros((16,), jnp.int32))
                bh = [exb] * 8
                tail = exb
            else:
                bh = [_vgather(ex, jnp.full((16,), hh, jnp.int32))
                      for hh in range(8)]
                tail = ex
            mrow = msg2.at[b].at[e]
            for j in range(4):
                w = rrow[pl.ds(16 * j, 16)]
                pa = lax.bitcast_convert_type(w << 16, jnp.float32)
                pb = lax.bitcast_convert_type(w & himask, jnp.float32)
                mrow[pl.ds(32 * j, 16)] = pa * bh[2 * j]
                mrow[pl.ds(32 * j + 16, 16)] = pb * bh[2 * j + 1]
            mrow[pl.ds(128, 16)] = tail

    fire_idx_block(0, 0)
    drain_idx_block(0)
    fire_gathers(0, srcB.at[0, 0], dstB.at[0, 0])
    fire_gathers(1, srcB.at[0, 1], dstB.at[0, 1])

    _NBLK = _NCHUNK // _IB

    def blockpair_body(i, carry):
        for bb in range(2):
            blk = 2 * i + bb
            nbb = bb ^ 1
            for k in range(_IB):
                b = k % 2
                ci = blk * _IB + k

                @pl.when(ci >= 2)
                def _():
                    drain_scatter(b)
                drain_gathers(b)
                compute_chunk(b)
                if k == 1:
                    @pl.when(blk + 1 < _NBLK)
                    def _():
                        fire_idx_block(nbb, blk + 1)
                if k == 2:
                    @pl.when(blk + 1 < _NBLK)
                    def _():
                        drain_idx_block(nbb)
                if k < _IB - 2:
                    fire_gathers(b, srcB.at[bb, k + 2], dstB.at[bb, k + 2])
                else:
                    @pl.when(blk + 1 < _NBLK)
                    def _():
                        fire_gathers(b, srcB.at[nbb, k + 2 - _IB],
                                     dstB.at[nbb, k + 2 - _IB])
                pltpu.async_copy(msg2.at[b], acc_sh.at[dstB.at[bb, k]],
                                 ssems[b], add=True)
        return carry

    lax.fori_loop(0, _NBLK // 2, blockpair_body, 0)
    for b in range(2):
        drain_scatter(b)
    plsc.subcore_barrier()
    pltpu.sync_copy(acc_sh.at[pl.ds(s * _RPT, _RPT)],
                    out_hbm.at[c, pl.ds(s * _RPT, _RPT)])


def _make_edge_pass(H):
    mesh = plsc.VectorSubcoreMesh(core_axis_name="c", subcore_axis_name="s")
    return functools.partial(
        pl.kernel,
        mesh=mesh,
        compiler_params=pltpu.CompilerParams(use_tc_tiling_on_sc=False),
        out_type=jax.ShapeDtypeStruct((_NC, _NACC, _ACC_W), jnp.float32),
        scratch_types=[
            pltpu.VMEM((2, _IB, _CH), jnp.int32),
            pltpu.VMEM((2, _IB, _CH), jnp.int32),
            pltpu.VMEM((2, _CH, 16), jnp.float32),
            pltpu.VMEM((2, _CH, 80), jnp.int32),
            pltpu.VMEM((2, _CH, _ACC_W), jnp.float32),
            pltpu.VMEM((16,), jnp.float32),
            pltpu.VMEM_SHARED((_NACC, _ACC_W), jnp.float32),
            pltpu.SemaphoreType.DMA,
            pltpu.SemaphoreType.DMA,
            pltpu.SemaphoreType.DMA,
            pltpu.SemaphoreType.DMA,
            pltpu.SemaphoreType.DMA,
        ],
    )(functools.partial(_edge_body, H))


_edge_pass_h8 = _make_edge_pass(8)
_edge_pass_h1 = _make_edge_pass(1)


def _dense_mid_body(a0, a1, c2w, m2s, m2d, rmat, c1b, pmat, qmat,
                    xw_o, sdd_o, bnd_o):
    i = pl.program_id(0)
    nsteps = pl.num_programs(0)
    u = a0[:, :128] + a1[:, :128]
    den = a0[:, 128:136] + a1[:, 128:136]
    rep = (1.0 / (den + 1e-16)) @ rmat[...]
    ef = u * rep + c1b[...]
    ef = jnp.where(ef > 0.0, ef, jnp.exp(ef) - 1.0)
    xw = ef @ c2w[...]
    sds = xw @ m2s[...]
    sdd = xw @ m2d[...]
    xw_o[...] = jnp.concatenate(
        [xw @ pmat[...], sds @ qmat[...]], axis=1).astype(jnp.bfloat16)
    sdd_o[...] = sdd
    v = jnp.concatenate([jnp.max(sds, axis=0, keepdims=True)[:, :1],
                         jnp.max(sdd, axis=0, keepdims=True)[:, :1],
                         jnp.zeros((1, 14), jnp.float32)], axis=1)
    vb = jnp.broadcast_to(v, (8, 16))

    @pl.when(i == 0)
    def _():
        bnd_o[...] = jnp.full((8, 16), -_BIG, jnp.float32)

    bnd_o[...] = jnp.maximum(bnd_o[...], vb)

    @pl.when(i == nsteps - 1)
    def _():
        cur = bnd_o[...]
        b0 = _lrelu(cur[:, 0:1] + cur[:, 1:2])
        bnd_o[...] = jnp.concatenate(
            [b0, jnp.full((8, 15), _BIG, jnp.float32)], axis=1)


def _dense_mid(a0, a1, c2w, m2s, m2d, rmat, c1b, pmat, qmat):
    nb = _N // _BN
    full = lambda shape: pl.BlockSpec(shape, lambda i: (0, 0))
    blk = lambda w: pl.BlockSpec((_BN, w), lambda i: (i, 0))
    return pl.pallas_call(
        _dense_mid_body,
        grid=(nb,),
        in_specs=[blk(_ACC_W), blk(_ACC_W), full((128, 128)),
                  full((128, 16)), full((128, 16)), full((8, 128)),
                  full((1, 128)), full((128, 128)), full((16, 32))],
        out_specs=[blk(160), blk(16),
                   pl.BlockSpec((8, 16), lambda i: (0, 0))],
        out_shape=[
            jax.ShapeDtypeStruct((_N, 160), jnp.bfloat16),
            jax.ShapeDtypeStruct((_N, 16), jnp.float32),
            jax.ShapeDtypeStruct((8, 16), jnp.float32),
        ],
    )(a0, a1, c2w, m2s, m2d, rmat, c1b, pmat, qmat)


def _dense_fin_body(a0, a1, lbl, emb2p, c2b, feat_o, oe2_o):
    den = a0[:, 128:129] + a1[:, 128:129]
    feat = (a0[:, :128] + a1[:, :128]) * (1.0 / (den + 1e-16)) + c2b[...]
    feat_o[...] = feat
    lblv = lbl[...]
    iota = lax.broadcasted_iota(jnp.int32, (_BN, 160), 1)
    p = (iota == lblv).astype(jnp.float32)
    oe2_o[...] = p @ emb2p[...]


def _dense_fin(a0, a1, lbl2, emb2p, c2b):
    nb = _N // _BN
    full = lambda shape: pl.BlockSpec(shape, lambda i: (0, 0))
    blk = lambda w: pl.BlockSpec((_BN, w), lambda i: (i, 0))
    return pl.pallas_call(
        _dense_fin_body,
        grid=(nb,),
        in_specs=[blk(_ACC_W), blk(_ACC_W), blk(1), full((160, _EMBED)),
                  full((1, 128))],
        out_specs=[blk(128), blk(_EMBED)],
        out_shape=[
            jax.ShapeDtypeStruct((_N, 128), jnp.float32),
            jax.ShapeDtypeStruct((_N, _EMBED), jnp.float32),
        ],
    )(a0, a1, lbl2, emb2p, c2b)


def kernel(roi_features, obj_labels, pos_input, edge_index, emb1, emb2,
           bb_w1, bb_b1, bb_w2, bb_b2,
           c1_W, c1_asrc, c1_adst, c1_b,
           c2_W, c2_asrc, c2_adst, c2_b):
    f32 = jnp.float32
    lbl = obj_labels.astype(jnp.int32)
    lbl2 = lbl.reshape(_N, 1)
    posp = jnp.pad(pos_input, ((0, 0), (0, 7)))
    emb1p = jnp.pad(emb1, ((0, 160 - _NUM_OBJ), (0, 0)))
    emb2p = jnp.pad(emb2, ((0, 160 - _NUM_OBJ), (0, 0)))
    w1p = jnp.pad(bb_w1, ((0, 7), (0, 0)))
    b1r = bb_b1.reshape(1, 32)
    b2r = bb_b2.reshape(1, 128)
    c1br = c1_b.reshape(1, 128)
    c2br = c2_b.reshape(1, 128)

    ms1 = jnp.zeros((128, 16), f32)
    md1 = jnp.zeros((128, 16), f32)
    for h in range(8):
        ms1 = ms1.at[h * 16:(h + 1) * 16, h].set(c1_asrc[0, h])
        md1 = md1.at[h * 16:(h + 1) * 16, h].set(c1_adst[0, h])
    m2s = jnp.zeros((128, 16), f32).at[:, 0].set(c2_asrc[0, 0])
    m2d = jnp.zeros((128, 16), f32).at[:, 0].set(c2_adst[0, 0])
    rmat = jnp.zeros((8, 128), f32)
    for h in range(8):
        rmat = rmat.at[h, h * 16:(h + 1) * 16].set(1.0)
    ar = jnp.arange(16)
    pmat = jnp.zeros((128, 128), f32)
    qmat = jnp.zeros((16, 32), f32).at[ar, 2 * ar].set(1.0)
    for j in range(4):
        pmat = pmat.at[32 * j + ar, 32 * j + 2 * ar].set(1.0)
        pmat = pmat.at[32 * j + 16 + ar, 32 * j + 2 * ar + 1].set(1.0)

    xwrow1, sdd1, dists, bnd1 = _dense_pre(
        roi_features, lbl2, posp, c1_W, emb1p, w1p, b1r, bb_w2, b2r, ms1, md1,
        pmat, qmat)

    eip = jnp.pad(edge_index.astype(jnp.int32), ((0, 0), (0, _EP - _E)),
                  constant_values=_N)
    src = eip[0].reshape(_EP // _CH, _CH)
    dst = eip[1].reshape(_EP // _CH, _CH)
    zeros_acc = jnp.zeros((_NACC, _ACC_W), f32)
    padn = lambda x: jnp.pad(x, ((0, _NACC - _N), (0, 0)))
    as_i32 = lambda x: lax.bitcast_convert_type(
        x.reshape(_N, 80, 2), jnp.int32)

    acc1 = _edge_pass_h8(src, dst, padn(as_i32(xwrow1)), padn(sdd1),
                         bnd1[0], zeros_acc)

    xwrow2, sdd2, bnd2 = _dense_mid(
        acc1[0, :_N], acc1[1, :_N], c2_W, m2s, m2d, rmat, c1br, pmat, qmat)

    acc2 = _edge_pass_h1(src, dst, padn(as_i32(xwrow2)), padn(sdd2),
                         bnd2[0], zeros_acc)

    obj_feats, oe2 = _dense_fin(acc2[0, :_N], acc2[1, :_N], lbl2, emb2p, c2br)

    edge_pre_rep = jnp.concatenate([roi_features, obj_feats, oe2], axis=-1)
    return dists, obj_labels, edge_pre_rep


# speedup vs baseline: 1.0049x; 1.0049x over previous
import functools

import jax
import jax.numpy as jnp
from jax import lax
from jax.experimental import pallas as pl
from jax.experimental.pallas import tpu as pltpu
from jax.experimental.pallas import tpu_sc as plsc

_N = 10000
_E = 640000
_NUM_OBJ = 151
_EMBED = 200
_ACC_W = 144
_NACC = 10016
_CH = 64
_NC = 2
_NS = 16
_NW = _NC * _NS
_NCHUNK = 320
_IB = 4
_EPW = _NCHUNK * _CH
_EP = _EPW * _NW
_RPT = _NACC // _NS
_BN = 1000
_BIG = 1e30


def _lrelu(x):
    return jnp.where(x >= 0.0, x, 0.2 * x)


def _vgather(x, idx):
    dn = lax.GatherDimensionNumbers(offset_dims=(), collapsed_slice_dims=(0,),
                                    start_index_map=(0,))
    return lax.gather(x, idx[:, None], dn, slice_sizes=(1,),
                      mode=lax.GatherScatterMode.PROMISE_IN_BOUNDS)


def _dense_pre_body(roi, lbl, pos, c1w, emb1p, w1, b1, w2, b2, ms, md,
                    pmat, qmat, xw_o, sdd_o, dist_o, bnd_o):
    i = pl.program_id(0)
    nsteps = pl.num_programs(0)
    lblv = lbl[...]
    iota = lax.broadcasted_iota(jnp.int32, (_BN, 160), 1)
    p = (iota == lblv).astype(jnp.float32)
    dist_o[...] = p[:, :_NUM_OBJ] * 2000.0 - 1000.0
    h = jnp.maximum(pos[...] @ w1[...] + b1[...], 0.0)
    pe = jnp.maximum(h @ w2[...] + b2[...], 0.0)
    cw = c1w[...]
    m1 = emb1p[...] @ cw[128:328]
    xw = roi[...] @ cw[:128] + p @ m1 + pe @ cw[328:456]
    sds = xw @ ms[...]
    sdd = xw @ md[...]
    xw_o[...] = jnp.concatenate(
        [xw @ pmat[...], sds @ qmat[...]], axis=1).astype(jnp.bfloat16)
    sdd_o[...] = sdd
    v = jnp.concatenate([jnp.max(sds, axis=0, keepdims=True)[:, :8],
                         jnp.max(sdd, axis=0, keepdims=True)[:, :8]], axis=1)
    vb = jnp.broadcast_to(v, (8, 16))

    @pl.when(i == 0)
    def _():
        bnd_o[...] = jnp.full((8, 16), -_BIG, jnp.float32)

    bnd_o[...] = jnp.maximum(bnd_o[...], vb)

    @pl.when(i == nsteps - 1)
    def _():
        cur = bnd_o[...]
        ssum = cur[:, :8] + cur[:, 8:16]
        b = _lrelu(ssum)
        bnd_o[...] = jnp.concatenate(
            [b, jnp.full((8, 8), _BIG, jnp.float32)], axis=1)


def _dense_pre(roi, lbl2, posp, c1w, emb1p, w1p, b1r, w2r, b2r, ms, md,
               pmat, qmat):
    nb = _N // _BN
    full = lambda shape: pl.BlockSpec(shape, lambda i: (0, 0))
    blk = lambda w: pl.BlockSpec((_BN, w), lambda i: (i, 0))
    return pl.pallas_call(
        _dense_pre_body,
        grid=(nb,),
        in_specs=[blk(128), blk(1), blk(16), full((456, 128)),
                  full((160, 128 + 72)), full((16, 32)), full((1, 32)),
                  full((32, 128)), full((1, 128)), full((128, 16)),
                  full((128, 16)), full((128, 128)), full((16, 32))],
        out_specs=[blk(160), blk(16), blk(_NUM_OBJ),
                   pl.BlockSpec((8, 16), lambda i: (0, 0))],
        out_shape=[
            jax.ShapeDtypeStruct((_N, 160), jnp.bfloat16),
            jax.ShapeDtypeStruct((_N, 16), jnp.float32),
            jax.ShapeDtypeStruct((_N, _NUM_OBJ), jnp.float32),
            jax.ShapeDtypeStruct((8, 16), jnp.float32),
        ],
    )(roi, lbl2, posp, c1w, emb1p, w1p, b1r, w2r, b2r, ms, md, pmat, qmat)


def _edge_body(H, src_hbm, dst_hbm, xw_hbm, sdd_hbm, bnd_hbm,
               zeros_hbm, out_hbm, srcB, dstB, gd2, rows2, msg2,
               bnd_v, acc_sh, gsem0, gsem1, ssem0, ssem1, isem):
    c = lax.axis_index("c")
    s = lax.axis_index("s")
    wid = c * _NS + s
    base = wid * _EPW
    gsems = [gsem0, gsem1]
    ssems = [ssem0, ssem1]
    pltpu.sync_copy(zeros_hbm.at[pl.ds(s * _RPT, _RPT)],
                    acc_sh.at[pl.ds(s * _RPT, _RPT)])
    pltpu.sync_copy(bnd_hbm, bnd_v)
    plsc.subcore_barrier()
    bvec = bnd_v[...]

    def fire_idx_block(bb, blk):
        row0 = wid * _NCHUNK + blk * _IB
        pltpu.async_copy(src_hbm.at[pl.ds(row0, _IB)], srcB.at[bb], isem)
        pltpu.async_copy(dst_hbm.at[pl.ds(row0, _IB)], dstB.at[bb], isem)

    def drain_idx_block(bb):
        pltpu.make_async_copy(src_hbm.at[pl.ds(0, _IB)], srcB.at[bb],
                              isem).wait()
        pltpu.make_async_copy(dst_hbm.at[pl.ds(0, _IB)], dstB.at[bb],
                              isem).wait()

    def fire_gathers(b, sidx, didx):
        pltpu.async_copy(xw_hbm.at[sidx], rows2.at[b], gsems[b])
        pltpu.async_copy(sdd_hbm.at[didx], gd2.at[b], gsems[b])

    def drain_gathers(b):
        pltpu.make_async_copy(xw_hbm.at[pl.ds(0, _CH)], rows2.at[b],
                              gsems[b]).wait()
        pltpu.make_async_copy(sdd_hbm.at[pl.ds(0, _CH)], gd2.at[b],
                              gsems[b]).wait()

    def drain_scatter(b):
        pltpu.make_async_copy(zeros_hbm.at[pl.ds(0, _CH)], msg2.at[b],
                              ssems[b]).wait()

    def compute_chunk(b):
        himask = jnp.full((16,), -65536, jnp.int32)

        @plsc.parallel_loop(0, _CH, step=1, unroll=4)
        def edge_body(e):
            rrow = rows2.at[b].at[e]
            wsd = rrow[pl.ds(64, 16)]
            gs = lax.bitcast_convert_type(wsd << 16, jnp.float32)
            a = gs + gd2.at[b][e]
            ex = jnp.exp(_lrelu(a) - bvec)
            if H == 1:
                exb = _vgather(ex, jnp.zeros((16,), jnp.int32))
                bh = [exb] * 8
                tail = exb
            else:
                bh = [_vgather(ex, jnp.full((16,), hh, jnp.int32))
                      for hh in range(8)]
                tail = ex
            mrow = msg2.at[b].at[e]
            for j in range(4):
                w = rrow[pl.ds(16 * j, 16)]
                pa = lax.bitcast_convert_type(w << 16, jnp.float32)
                pb = lax.bitcast_convert_type(w & himask, jnp.float32)
                mrow[pl.ds(32 * j, 16)] = pa * bh[2 * j]
                mrow[pl.ds(32 * j + 16, 16)] = pb * bh[2 * j + 1]
            mrow[pl.ds(128, 16)] = tail

    fire_idx_block(0, 0)
    drain_idx_block(0)
    fire_gathers(0, srcB.at[0, 0], dstB.at[0, 0])
    fire_gathers(1, srcB.at[0, 1], dstB.at[0, 1])

    _NBLK = _NCHUNK // _IB

    def blockpair_body(i, carry):
        for bb in range(2):
            blk = 2 * i + bb
            nbb = bb ^ 1
            for k in range(_IB):
                b = k % 2
                ci = blk * _IB + k

                @pl.when(ci >= 2)
                def _():
                    drain_scatter(b)
                drain_gathers(b)
                compute_chunk(b)
                if k == 1:
                    @pl.when(blk + 1 < _NBLK)
                    def _():
                        fire_idx_block(nbb, blk + 1)
                if k == 2:
                    @pl.when(blk + 1 < _NBLK)
                    def _():
                        drain_idx_block(nbb)
                if k < _IB - 2:
                    fire_gathers(b, srcB.at[bb, k + 2], dstB.at[bb, k + 2])
                else:
                    @pl.when(blk + 1 < _NBLK)
                    def _():
                        fire_gathers(b, srcB.at[nbb, k + 2 - _IB],
                                     dstB.at[nbb, k + 2 - _IB])
                pltpu.async_copy(msg2.at[b], acc_sh.at[dstB.at[bb, k]],
                                 ssems[b], add=True)
        return carry

    lax.fori_loop(0, _NBLK // 2, blockpair_body, 0)
    for b in range(2):
        drain_scatter(b)
    plsc.subcore_barrier()
    pltpu.sync_copy(acc_sh.at[pl.ds(s * _RPT, _RPT)],
                    out_hbm.at[c, pl.ds(s * _RPT, _RPT)])


def _make_edge_pass(H):
    mesh = plsc.VectorSubcoreMesh(core_axis_name="c", subcore_axis_name="s")
    return functools.partial(
        pl.kernel,
        mesh=mesh,
        compiler_params=pltpu.CompilerParams(use_tc_tiling_on_sc=False),
        out_type=jax.ShapeDtypeStruct((_NC, _NACC, _ACC_W), jnp.float32),
        scratch_types=[
            pltpu.VMEM((2, _IB, _CH), jnp.int32),
            pltpu.VMEM((2, _IB, _CH), jnp.int32),
            pltpu.VMEM((2, _CH, 16), jnp.float32),
            pltpu.VMEM((2, _CH, 80), jnp.int32),
            pltpu.VMEM((2, _CH, _ACC_W), jnp.float32),
            pltpu.VMEM((16,), jnp.float32),
            pltpu.VMEM_SHARED((_NACC, _ACC_W), jnp.float32),
            pltpu.SemaphoreType.DMA,
            pltpu.SemaphoreType.DMA,
            pltpu.SemaphoreType.DMA,
            pltpu.SemaphoreType.DMA,
            pltpu.SemaphoreType.DMA,
        ],
    )(functools.partial(_edge_body, H))


_edge_pass_h8 = _make_edge_pass(8)
_edge_pass_h1 = _make_edge_pass(1)


def _dense_mid_body(a0, a1, c2w, m2s, m2d, rmat, c1b, pmat, qmat,
                    xw_o, sdd_o, bnd_o):
    i = pl.program_id(0)
    nsteps = pl.num_programs(0)
    u = a0[:, :128] + a1[:, :128]
    den = a0[:, 128:136] + a1[:, 128:136]
    rep = (1.0 / (den + 1e-16)) @ rmat[...]
    ef = u * rep + c1b[...]
    ef = jnp.where(ef > 0.0, ef, jnp.exp(ef) - 1.0)
    xw = ef @ c2w[...]
    sds = xw @ m2s[...]
    sdd = xw @ m2d[...]
    xw_o[...] = jnp.concatenate(
        [xw @ pmat[...], sds @ qmat[...]], axis=1).astype(jnp.bfloat16)
    sdd_o[...] = sdd
    v = jnp.concatenate([jnp.max(sds, axis=0, keepdims=True)[:, :1],
                         jnp.max(sdd, axis=0, keepdims=True)[:, :1],
                         jnp.zeros((1, 14), jnp.float32)], axis=1)
    vb = jnp.broadcast_to(v, (8, 16))

    @pl.when(i == 0)
    def _():
        bnd_o[...] = jnp.full((8, 16), -_BIG, jnp.float32)

    bnd_o[...] = jnp.maximum(bnd_o[...], vb)

    @pl.when(i == nsteps - 1)
    def _():
        cur = bnd_o[...]
        b0 = _lrelu(cur[:, 0:1] + cur[:, 1:2])
        bnd_o[...] = jnp.concatenate(
            [b0, jnp.full((8, 15), _BIG, jnp.float32)], axis=1)


def _dense_mid(a0, a1, c2w, m2s, m2d, rmat, c1b, pmat, qmat):
    nb = _N // _BN
    full = lambda shape: pl.BlockSpec(shape, lambda i: (0, 0))
    blk = lambda w: pl.BlockSpec((_BN, w), lambda i: (i, 0))
    return pl.pallas_call(
        _dense_mid_body,
        grid=(nb,),
        in_specs=[blk(_ACC_W), blk(_ACC_W), full((128, 128)),
                  full((128, 16)), full((128, 16)), full((8, 128)),
                  full((1, 128)), full((128, 128)), full((16, 32))],
        out_specs=[blk(160), blk(16),
                   pl.BlockSpec((8, 16), lambda i: (0, 0))],
        out_shape=[
            jax.ShapeDtypeStruct((_N, 160), jnp.bfloat16),
            jax.ShapeDtypeStruct((_N, 16), jnp.float32),
            jax.ShapeDtypeStruct((8, 16), jnp.float32),
        ],
    )(a0, a1, c2w, m2s, m2d, rmat, c1b, pmat, qmat)


def _dense_fin_body(a0, a1, lbl, emb2p, c2b, feat_o, oe2_o):
    den = a0[:, 128:129] + a1[:, 128:129]
    feat = (a0[:, :128] + a1[:, :128]) * (1.0 / (den + 1e-16)) + c2b[...]
    feat_o[...] = feat
    lblv = lbl[...]
    iota = lax.broadcasted_iota(jnp.int32, (_BN, 160), 1)
    p = (iota == lblv).astype(jnp.float32)
    oe2_o[...] = p @ emb2p[...]


def _dense_fin(a0, a1, lbl2, emb2p, c2b):
    nb = _N // _BN
    full = lambda shape: pl.BlockSpec(shape, lambda i: (0, 0))
    blk = lambda w: pl.BlockSpec((_BN, w), lambda i: (i, 0))
    return pl.pallas_call(
        _dense_fin_body,
        grid=(nb,),
        in_specs=[blk(_ACC_W), blk(_ACC_W), blk(1), full((160, _EMBED)),
                  full((1, 128))],
        out_specs=[blk(128), blk(_EMBED)],
        out_shape=[
            jax.ShapeDtypeStruct((_N, 128), jnp.float32),
            jax.ShapeDtypeStruct((_N, _EMBED), jnp.float32),
        ],
    )(a0, a1, lbl2, emb2p, c2b)


def kernel(roi_features, obj_labels, pos_input, edge_index, emb1, emb2,
           bb_w1, bb_b1, bb_w2, bb_b2,
           c1_W, c1_asrc, c1_adst, c1_b,
           c2_W, c2_asrc, c2_adst, c2_b):
    f32 = jnp.float32
    lbl = obj_labels.astype(jnp.int32)
    lbl2 = lbl.reshape(_N, 1)
    posp = jnp.pad(pos_input, ((0, 0), (0, 7)))
    emb1p = jnp.pad(emb1, ((0, 160 - _NUM_OBJ), (0, 0)))
    emb2p = jnp.pad(emb2, ((0, 160 - _NUM_OBJ), (0, 0)))
    w1p = jnp.pad(bb_w1, ((0, 7), (0, 0)))
    b1r = bb_b1.reshape(1, 32)
    b2r = bb_b2.reshape(1, 128)
    c1br = c1_b.reshape(1, 128)
    c2br = c2_b.reshape(1, 128)

    ms1 = jnp.zeros((128, 16), f32)
    md1 = jnp.zeros((128, 16), f32)
    for h in range(8):
        ms1 = ms1.at[h * 16:(h + 1) * 16, h].set(c1_asrc[0, h])
        md1 = md1.at[h * 16:(h + 1) * 16, h].set(c1_adst[0, h])
    m2s = jnp.zeros((128, 16), f32).at[:, 0].set(c2_asrc[0, 0])
    m2d = jnp.zeros((128, 16), f32).at[:, 0].set(c2_adst[0, 0])
    rmat = jnp.zeros((8, 128), f32)
    for h in range(8):
        rmat = rmat.at[h, h * 16:(h + 1) * 16].set(1.0)
    ar = jnp.arange(16)
    pmat = jnp.zeros((128, 128), f32)
    qmat = jnp.zeros((16, 32), f32).at[ar, 2 * ar].set(1.0)
    for j in range(4):
        pmat = pmat.at[32 * j + ar, 32 * j + 2 * ar].set(1.0)
        pmat = pmat.at[32 * j + 16 + ar, 32 * j + 2 * ar + 1].set(1.0)

    xwrow1, sdd1, dists, bnd1 = _dense_pre(
        roi_features, lbl2, posp, c1_W, emb1p, w1p, b1r, bb_w2, b2r, ms1, md1,
        pmat, qmat)

    eip = jnp.pad(edge_index.astype(jnp.int32), ((0, 0), (0, _EP - _E)),
                  constant_values=_N)
    src = eip[0].reshape(_EP // _CH, _CH)
    dst = eip[1].reshape(_EP // _CH, _CH)
    zeros_acc = jnp.zeros((_NACC, _ACC_W), f32)
    padn = lambda x: jnp.pad(x, ((0, _NACC - _N), (0, 0)))
    as_i32 = lambda x: lax.bitcast_convert_type(
        x.reshape(_N, 80, 2), jnp.int32)

    acc1 = _edge_pass_h8(src, dst, padn(as_i32(xwrow1)), padn(sdd1),
                         bnd1[0], zeros_acc)

    xwrow2, sdd2, bnd2 = _dense_mid(
        acc1[0, :_N], acc1[1, :_N], c2_W, m2s, m2d, rmat, c1br, pmat, qmat)

    acc2 = _edge_pass_h1(src, dst, padn(as_i32(xwrow2)), padn(sdd2),
                         bnd2[0], zeros_acc)

    obj_feats, oe2 = _dense_fin(acc2[0, :_N], acc2[1, :_N], lbl2, emb2p, c2br)

    edge_pre_rep = jnp.concatenate([roi_features, obj_feats, oe2], axis=-1)
    return dists, obj_labels, edge_pre_rep
